# Initial kernel scaffold; baseline (speedup 1.0000x reference)
import jax
import jax.numpy as jnp
from jax.experimental import pallas as pl


def kernel(x, codebook):
    raise NotImplementedError("write your pallas kernel here")


# speedup vs baseline: 1.5432x; 1.5432x over previous
import functools

import jax
import jax.numpy as jnp
from jax.experimental import pallas as pl
from jax.experimental.pallas import tpu as pltpu
from jax.experimental.pallas import tpu_sc as plsc

_BN = 256

_PREC = jax.lax.Precision.DEFAULT


def _assign_body(x_ref, cbt_ref, enc_ref, b2_ref):
    i = pl.program_id(0)
    k = cbt_ref.shape[1]

    @pl.when(i == 0)
    def _():
        cbt = cbt_ref[...]
        b2_ref[...] = jnp.sum(cbt * cbt, axis=0, keepdims=True)

    x = x_ref[...]
    a2 = jnp.sum(x * x, axis=1, keepdims=True)
    s = jax.lax.dot_general(
        x, cbt_ref[...], (((1,), (0,)), ((), ())),
        preferred_element_type=jnp.float32, precision=_PREC,
    )
    d2 = (a2 - 2.0 * s) + b2_ref[...]
    m = jnp.min(d2, axis=1, keepdims=True)
    thr = jnp.where(m > 0, m * (1.0 + 0.9e-7), 0.0)
    iota = jax.lax.broadcasted_iota(jnp.int32, d2.shape, 1)
    idx = jnp.min(jnp.where(d2 <= thr, iota, k), axis=1, keepdims=True)
    enc_ref[...] = idx


@functools.partial(jax.jit, static_argnums=())
def _assign(x, cbt):
    n, d = x.shape
    k = cbt.shape[1]
    return pl.pallas_call(
        _assign_body,
        grid=(n // _BN,),
        in_specs=[
            pl.BlockSpec((_BN, d), lambda i: (i, 0)),
            pl.BlockSpec((d, k), lambda i: (0, 0)),
        ],
        out_specs=pl.BlockSpec((_BN, 1), lambda i: (i, 0)),
        out_shape=jax.ShapeDtypeStruct((n, 1), jnp.int32),
        scratch_shapes=[pltpu.VMEM((1, k), jnp.float32)],
    )(x, cbt)


_GATHER_W = 128


def _sc_gather(codebook, idx2d):
    n = idx2d.shape[1]
    d = codebook.shape[1]
    mesh = plsc.VectorSubcoreMesh(core_axis_name="c", subcore_axis_name="s")

    @functools.partial(
        pl.kernel,
        out_type=jax.ShapeDtypeStruct((n, d), codebook.dtype),
        mesh=mesh,
    )
    def _gather_kernel(cb_hbm, i_hbm, o_hbm):
        def body(i_vmem, o_vmem):
            pltpu.sync_copy(cb_hbm.at[i_vmem.at[0]], o_vmem)

        pltpu.emit_pipeline(
            body,
            grid=(n // _GATHER_W,),
            in_specs=[pl.BlockSpec((1, _GATHER_W), lambda i: (0, i))],
            out_specs=[pl.BlockSpec((_GATHER_W, d), lambda i: (i, 0))],
            core_axis_name=("c", "s"),
            dimension_semantics=(pltpu.PARALLEL,),
        )(i_hbm, o_hbm)

    return _gather_kernel(codebook, idx2d)


def kernel(x, codebook):
    n = x.shape[0]
    cbt = codebook.T
    enc = _assign(x, cbt)
    idx2d = enc.reshape(1, n)
    quantized = _sc_gather(codebook, idx2d)
    return (enc.reshape(n), quantized)
